# baseline (speedup 1.0000x reference)
import functools

import jax
import jax.numpy as jnp
from jax.experimental import pallas as pl
from jax.experimental.pallas import tpu as pltpu

_BM, _BN, _BK = 2048, 2048, 512
_INV_SQRT2 = 0.7071067811865476


def _matmul_gelu_kernel(a_ref, w_ref, b_ref, o_ref, *, k_steps):
    if k_steps == 1:
        x = jnp.dot(a_ref[...], w_ref[...],
                    preferred_element_type=jnp.float32) + b_ref[...]
        o_ref[...] = x * (0.5 * (1.0 + jax.lax.erf(x * _INV_SQRT2)))
        return

    k = pl.program_id(2)

    @pl.when(k == 0)
    def _first():
        o_ref[...] = jnp.dot(a_ref[...], w_ref[...],
                             preferred_element_type=jnp.float32) + b_ref[...]

    @pl.when(jnp.logical_and(k > 0, k < k_steps - 1))
    def _middle():
        o_ref[...] += jnp.dot(a_ref[...], w_ref[...],
                              preferred_element_type=jnp.float32)

    @pl.when(k == k_steps - 1)
    def _finish():
        x = o_ref[...] + jnp.dot(a_ref[...], w_ref[...],
                                 preferred_element_type=jnp.float32)
        u = 0.5 * x
        o_ref[...] = u * jax.lax.erf(x * _INV_SQRT2) + u


def kernel(hidden_states, W, b):
    batch, seq, d_in = hidden_states.shape
    m = batch * seq
    k_dim, n = W.shape
    a = hidden_states.reshape(m, d_in)
    b2 = b.reshape(1, n)

    bm, bn, bk = min(_BM, m), min(_BN, n), min(_BK, k_dim)
    k_steps = k_dim // bk
    grid = (m // bm, n // bn, k_steps)

    out = pl.pallas_call(
        functools.partial(_matmul_gelu_kernel, k_steps=k_steps),
        grid=grid,
        in_specs=[
            pl.BlockSpec((bm, bk), lambda mi, ni, ki: (mi, ki)),
            pl.BlockSpec((bk, bn), lambda mi, ni, ki: (ki, ni)),
            pl.BlockSpec((1, bn), lambda mi, ni, ki: (0, ni)),
        ],
        out_specs=pl.BlockSpec((bm, bn), lambda mi, ni, ki: (mi, ni)),
        out_shape=jax.ShapeDtypeStruct((m, n), jnp.float32),
        compiler_params=pltpu.CompilerParams(
            dimension_semantics=("parallel", "parallel", "arbitrary"),
        ),
    )(a, W, b2)
    return out.reshape(batch, seq, n)
